# baseline (speedup 1.0000x reference)
import jax
import jax.numpy as jnp
from jax import lax
from jax.experimental import pallas as pl
from jax.experimental.pallas import tpu as pltpu
from jax.experimental.pallas import tpu_sc as plsc

_NC, _NS, _L = 2, 16, 16
_NW = _NC * _NS
_B = 16384
_D = 32
_BPW = _B // _NW
_G = _BPW // _L
_GW = 16
_RPG = _D * _L
_RPC = 1000000 // _GW


def _body(uids, iids, uembg, iembg, out, uid_v, iid_v,
          ugr0, ugr1, igr0, igr1, uix0, uix1, iix0, iix1, out_v,
          su0, su1, si0, si1):
    wid = lax.axis_index("s") * _NC + lax.axis_index("c")
    base = wid * _BPW
    pltpu.sync_copy(uids.at[pl.ds(base, _BPW)], uid_v)
    pltpu.sync_copy(iids.at[pl.ds(base, _BPW)], iid_v)

    iota = lax.iota(jnp.int32, _L)

    def issue(g, uix, iix, ugr, igr, su, si):
        uvec = uid_v[pl.ds(g * _L, _L)]
        ivec = iid_v[pl.ds(g * _L, _L)]
        ubase = uvec >> 4
        ibase = ivec >> 4
        for c in range(_D):
            pltpu.async_copy(
                uembg.at[ubase + c * _RPC],
                ugr.at[pl.ds(c * _L, _L)], su)
            pltpu.async_copy(
                iembg.at[ibase + c * _RPC],
                igr.at[pl.ds(c * _L, _L)], si)
        for c in range(_D):
            pltpu.make_async_copy(
                uembg.at[ubase], ugr.at[pl.ds(c * _L, _L)], su).wait()
            pltpu.make_async_copy(
                iembg.at[ibase], igr.at[pl.ds(c * _L, _L)], si).wait()

    def compute(g, ugr, igr):
        uvec = uid_v[pl.ds(g * _L, _L)]
        ivec = iid_v[pl.ds(g * _L, _L)]
        ulow = uvec & (_GW - 1)
        ilow = ivec & (_GW - 1)
        acc = jnp.zeros((_L,), jnp.float32)
        for c in range(_D):
            rows = c * _L + iota
            uv = plsc.load_gather(ugr, [rows, ulow])
            iv = plsc.load_gather(igr, [rows, ilow])
            acc = acc + uv * iv
        out_v[pl.ds(g * _L, _L)] = acc

    def step(g, carry):
        issue(g, uix0, iix0, ugr0, igr0, su0, si0)
        compute(g, ugr0, igr0)
        return carry

    lax.fori_loop(0, _G, step, 0)
    pltpu.sync_copy(out_v, out.at[pl.ds(base, _BPW)])


def kernel(user_ids, item_ids, user_emb, item_emb):
    mesh = plsc.VectorSubcoreMesh(
        core_axis_name="c", subcore_axis_name="s",
        num_cores=_NC, num_subcores=_NS)
    f = pl.kernel(
        _body,
        out_type=jax.ShapeDtypeStruct((_B,), jnp.float32),
        mesh=mesh,
        scratch_types=[
            pltpu.VMEM((_BPW,), jnp.int32),
            pltpu.VMEM((_BPW,), jnp.int32),
            pltpu.VMEM((_RPG, _GW), jnp.float32),
            pltpu.VMEM((_RPG, _GW), jnp.float32),
            pltpu.VMEM((_RPG, _GW), jnp.float32),
            pltpu.VMEM((_RPG, _GW), jnp.float32),
            pltpu.VMEM((4, 128), jnp.int32),
            pltpu.VMEM((4, 128), jnp.int32),
            pltpu.VMEM((4, 128), jnp.int32),
            pltpu.VMEM((4, 128), jnp.int32),
            pltpu.VMEM((_BPW,), jnp.float32),
            pltpu.SemaphoreType.DMA,
            pltpu.SemaphoreType.DMA,
            pltpu.SemaphoreType.DMA,
            pltpu.SemaphoreType.DMA,
        ],
        compiler_params=pltpu.CompilerParams(
            needs_layout_passes=False, use_tc_tiling_on_sc=False,
            disable_bounds_checks=True),
    )
    uembg = user_emb.T.reshape(_D * _RPC, _GW)
    iembg = item_emb.T.reshape(_D * _RPC, _GW)
    return f(user_ids, item_ids, uembg, iembg)


# speedup vs baseline: 1.0040x; 1.0040x over previous
import jax
import jax.numpy as jnp
from jax import lax
from jax.experimental import pallas as pl
from jax.experimental.pallas import tpu as pltpu
from jax.experimental.pallas import tpu_sc as plsc

_NC, _NS, _L = 2, 16, 16
_NW = _NC * _NS
_B = 16384
_D = 32
_BPW = _B // _NW
_LPP = 32
_NPH = _BPW // _LPP
_GPP = _LPP // _L
_GW = 16
_RPP = _D * _LPP
_RPC = 1000000 // _GW


def _body(uids, iids, uembg, iembg, out, uid_v, iid_v,
          ugr0, ugr1, igr0, igr1, uix0, uix1, iix0, iix1, out_v,
          su0, su1, si0, si1):
    wid = lax.axis_index("s") * _NC + lax.axis_index("c")
    base = wid * _BPW
    pltpu.sync_copy(uids.at[pl.ds(base, _BPW)], uid_v)
    pltpu.sync_copy(iids.at[pl.ds(base, _BPW)], iid_v)

    iota = lax.iota(jnp.int32, _L)

    def build_idx(ph, uix, iix):
        def grp(j, carry):
            uvec = uid_v[pl.ds(ph * _LPP + j * _L, _L)]
            ivec = iid_v[pl.ds(ph * _LPP + j * _L, _L)]
            ubase = uvec >> 4
            ibase = ivec >> 4
            for c in range(_D):
                uix[pl.ds((c * _GPP + j) * _L, _L)] = ubase + c * _RPC
                iix[pl.ds((c * _GPP + j) * _L, _L)] = ibase + c * _RPC
            return carry
        lax.fori_loop(0, _GPP, grp, 0)

    def compute(ph, ugr, igr):
        def grp(j, carry):
            uvec = uid_v[pl.ds(ph * _LPP + j * _L, _L)]
            ivec = iid_v[pl.ds(ph * _LPP + j * _L, _L)]
            ulow = uvec & (_GW - 1)
            ilow = ivec & (_GW - 1)
            acc = jnp.zeros((_L,), jnp.float32)
            for c in range(_D):
                rows = (c * _GPP + j) * _L + iota
                uv = plsc.load_gather(ugr, [rows, ulow])
                iv = plsc.load_gather(igr, [rows, ilow])
                acc = acc + uv * iv
            out_v[pl.ds(ph * _LPP + j * _L, _L)] = acc
            return carry
        lax.fori_loop(0, _GPP, grp, 0)

    bufs = ((ugr0, igr0, uix0, iix0, su0, si0),
            (ugr1, igr1, uix1, iix1, su1, si1))

    build_idx(0, uix0, iix0)
    pend = (pltpu.async_copy(uembg.at[uix0], ugr0, su0),
            pltpu.async_copy(iembg.at[iix0], igr0, si0))

    for ph in range(_NPH):
        nxt = None
        if ph + 1 < _NPH:
            ugr_n, igr_n, uix_n, iix_n, su_n, si_n = bufs[(ph + 1) % 2]
            build_idx(ph + 1, uix_n, iix_n)
            nxt = (pltpu.async_copy(uembg.at[uix_n], ugr_n, su_n),
                   pltpu.async_copy(iembg.at[iix_n], igr_n, si_n))
        pend[0].wait()
        pend[1].wait()
        compute(ph, bufs[ph % 2][0], bufs[ph % 2][1])
        pend = nxt

    pltpu.sync_copy(out_v, out.at[pl.ds(base, _BPW)])


def kernel(user_ids, item_ids, user_emb, item_emb):
    mesh = plsc.VectorSubcoreMesh(
        core_axis_name="c", subcore_axis_name="s",
        num_cores=_NC, num_subcores=_NS)
    f = pl.kernel(
        _body,
        out_type=jax.ShapeDtypeStruct((_B,), jnp.float32),
        mesh=mesh,
        scratch_types=[
            pltpu.VMEM((_BPW,), jnp.int32),
            pltpu.VMEM((_BPW,), jnp.int32),
            pltpu.VMEM((_RPP, _GW), jnp.float32),
            pltpu.VMEM((_RPP, _GW), jnp.float32),
            pltpu.VMEM((_RPP, _GW), jnp.float32),
            pltpu.VMEM((_RPP, _GW), jnp.float32),
            pltpu.VMEM((_RPP,), jnp.int32),
            pltpu.VMEM((_RPP,), jnp.int32),
            pltpu.VMEM((_RPP,), jnp.int32),
            pltpu.VMEM((_RPP,), jnp.int32),
            pltpu.VMEM((_BPW,), jnp.float32),
            pltpu.SemaphoreType.DMA,
            pltpu.SemaphoreType.DMA,
            pltpu.SemaphoreType.DMA,
            pltpu.SemaphoreType.DMA,
        ],
        compiler_params=pltpu.CompilerParams(
            needs_layout_passes=False, use_tc_tiling_on_sc=False,
            disable_bounds_checks=True),
    )
    uembg = user_emb.T.reshape(_D * _RPC, _GW)
    iembg = item_emb.T.reshape(_D * _RPC, _GW)
    return f(user_ids, item_ids, uembg, iembg)


# speedup vs baseline: 5.6515x; 5.6292x over previous
import jax
import jax.numpy as jnp
from jax import lax
from jax.experimental import pallas as pl
from jax.experimental.pallas import tpu as pltpu
from jax.experimental.pallas import tpu_sc as plsc

_NC, _NS, _L = 2, 16, 16
_NW = _NC * _NS
_B = 16384
_D = 32
_BPW = _B // _NW
_G = _BPW // _L


def _body(uids, iids, uemb, iemb, out, uid_v, iid_v, urow_v, irow_v, out_v,
          sem_u, sem_i):
    wid = lax.axis_index("s") * _NC + lax.axis_index("c")
    base = wid * _BPW
    pltpu.sync_copy(uids.at[pl.ds(base, _BPW)], uid_v)
    pltpu.sync_copy(iids.at[pl.ds(base, _BPW)], iid_v)
    cu = pltpu.async_copy(uemb.at[uid_v], urow_v, sem_u)
    ci = pltpu.async_copy(iemb.at[iid_v], irow_v, sem_i)
    cu.wait()
    ci.wait()

    iota = lax.iota(jnp.int32, _L)

    def group(g, carry):
        rows = g * _L + iota
        acc = jnp.zeros((_L,), jnp.float32)
        for c in range(_D):
            cols = jnp.full((_L,), c, jnp.int32)
            uv = plsc.load_gather(urow_v, [rows, cols])
            iv = plsc.load_gather(irow_v, [rows, cols])
            acc = acc + uv * iv
        out_v[pl.ds(g * _L, _L)] = acc
        return carry

    lax.fori_loop(0, _G, group, 0)
    pltpu.sync_copy(out_v, out.at[pl.ds(base, _BPW)])


def kernel(user_ids, item_ids, user_emb, item_emb):
    mesh = plsc.VectorSubcoreMesh(
        core_axis_name="c", subcore_axis_name="s",
        num_cores=_NC, num_subcores=_NS)
    f = pl.kernel(
        _body,
        out_type=jax.ShapeDtypeStruct((_B,), jnp.float32),
        mesh=mesh,
        scratch_types=[
            pltpu.VMEM((_BPW,), jnp.int32),
            pltpu.VMEM((_BPW,), jnp.int32),
            pltpu.VMEM((_BPW, _D), jnp.float32),
            pltpu.VMEM((_BPW, _D), jnp.float32),
            pltpu.VMEM((_BPW,), jnp.float32),
            pltpu.SemaphoreType.DMA,
            pltpu.SemaphoreType.DMA,
        ],
        compiler_params=pltpu.CompilerParams(
            needs_layout_passes=False, use_tc_tiling_on_sc=False),
    )
    return f(user_ids, item_ids, user_emb, item_emb)
